# Initial kernel scaffold; baseline (speedup 1.0000x reference)
import jax
import jax.numpy as jnp
from jax.experimental import pallas as pl


def kernel(x, edge_index, W1, b1, W2, b2, W3, b3):
    raise NotImplementedError("write your pallas kernel here")


# speedup vs baseline: 21.0522x; 21.0522x over previous
import functools

import jax
import jax.numpy as jnp
from jax import lax
from jax.experimental import pallas as pl
from jax.experimental.pallas import tpu as pltpu
from jax.experimental.pallas import tpu_sc as plsc

N = 10000
E = 320000
D_IN = 128
D = 64
NC = 2
NS = 16
NW = NC * NS
EPT = E // NW
STEP = 80
NSTEP = EPT // STEP
NPAD = 10240
RPT = NPAD // NS

_mesh = plsc.VectorSubcoreMesh(
    core_axis_name="c", subcore_axis_name="s", num_cores=NC, num_subcores=NS
)
_sc_params = pltpu.CompilerParams(use_tc_tiling_on_sc=False)


@functools.partial(
    pl.kernel,
    out_type=jax.ShapeDtypeStruct((NC, NPAD), jnp.float32),
    mesh=_mesh,
    compiler_params=_sc_params,
    scratch_types=[
        pltpu.VMEM((NSTEP, STEP), jnp.int32),
        pltpu.VMEM((RPT,), jnp.float32),
        pltpu.VMEM_SHARED((NPAD,), jnp.float32),
    ],
)
def _sc_degree(dst_hbm, out_hbm, idx_v, ones_v, hist_s):
    c = lax.axis_index("c")
    s = lax.axis_index("s")
    wid = s * NC + c
    r0 = s * RPT

    def fill(i, _):
        ones_v[pl.ds(i * 16, 16)] = jnp.full((16,), 1.0, jnp.float32)
        return 0

    lax.fori_loop(0, RPT // 16, fill, 0)
    pltpu.sync_copy(ones_v, hist_s.at[pl.ds(r0, RPT)])
    pltpu.sync_copy(dst_hbm.at[wid], idx_v)
    plsc.subcore_barrier()

    def body(j, _):
        pltpu.sync_copy(
            ones_v.at[pl.ds(0, STEP)], hist_s.at[idx_v.at[j]], add=True
        )
        return 0

    lax.fori_loop(0, NSTEP, body, 0)
    plsc.subcore_barrier()
    pltpu.sync_copy(hist_s.at[pl.ds(r0, RPT)], out_hbm.at[c, pl.ds(r0, RPT)])


@functools.partial(
    pl.kernel,
    out_type=jax.ShapeDtypeStruct((NC, NPAD, D), jnp.float32),
    mesh=_mesh,
    compiler_params=_sc_params,
    scratch_types=[
        pltpu.VMEM((NSTEP, STEP), jnp.int32),
        pltpu.VMEM((NSTEP, STEP), jnp.int32),
        pltpu.VMEM((STEP, D), jnp.float32),
        pltpu.VMEM((STEP, D), jnp.float32),
        pltpu.VMEM_SHARED((NPAD, D), jnp.float32),
        pltpu.SemaphoreType.DMA,
        pltpu.SemaphoreType.DMA,
    ],
)
def _sc_scatter(y_hbm, src_hbm, dst_hbm, out_hbm, si_v, di_v, b0, b1, acc_s,
                sem0, sem1):
    c = lax.axis_index("c")
    s = lax.axis_index("s")
    wid = s * NC + c
    r0 = s * RPT

    pltpu.sync_copy(y_hbm.at[pl.ds(r0, RPT)], acc_s.at[pl.ds(r0, RPT)])
    pltpu.sync_copy(src_hbm.at[wid], si_v)
    pltpu.sync_copy(dst_hbm.at[wid], di_v)
    plsc.subcore_barrier()

    def body(j, _):
        pltpu.sync_copy(y_hbm.at[si_v.at[j]], b0)
        pltpu.sync_copy(b0, acc_s.at[di_v.at[j]], add=True)
        return 0

    lax.fori_loop(0, NSTEP, body, 0)
    plsc.subcore_barrier()
    pltpu.sync_copy(acc_s.at[pl.ds(r0, RPT)], out_hbm.at[c, pl.ds(r0, RPT)])


def _dinv(deg2):
    return lax.rsqrt(deg2[:, 0:1] + deg2[:, 1:2] - 1.0)


def _tc_first(x_ref, w_ref, deg2_ref, y_ref):
    di = _dinv(deg2_ref[...])
    h = jnp.dot(x_ref[...], w_ref[...], preferred_element_type=jnp.float32)
    y_ref[:N, :] = h * di[:N]


def _tc_mid(sp_ref, y_ref, deg2_ref, b_ref, w_ref, o_ref):
    di = _dinv(deg2_ref[...])
    ssum = sp_ref[0] + sp_ref[1] - y_ref[...]
    z = jnp.maximum(ssum * di + b_ref[...], 0.0)
    h = jnp.dot(z[:N], w_ref[...], preferred_element_type=jnp.float32)
    o_ref[:N, :] = h * di[:N]


def _tc_last(sp_ref, y_ref, deg2_ref, b_ref, o_ref):
    di = _dinv(deg2_ref[...])
    ssum = sp_ref[0, :N] + sp_ref[1, :N] - y_ref[:N]
    o_ref[...] = jnp.maximum(ssum * di[:N] + b_ref[...], 0.0)


_t_first = pl.pallas_call(
    _tc_first, out_shape=jax.ShapeDtypeStruct((NPAD, D), jnp.float32)
)
_t_mid = pl.pallas_call(
    _tc_mid, out_shape=jax.ShapeDtypeStruct((NPAD, D), jnp.float32)
)
_t_last = pl.pallas_call(
    _tc_last, out_shape=jax.ShapeDtypeStruct((N, D), jnp.float32)
)


def kernel(x, edge_index, W1, b1, W2, b2, W3, b3):
    ei = edge_index.astype(jnp.int32)
    src = ei[0].reshape(NW, NSTEP, STEP)
    dst = ei[1].reshape(NW, NSTEP, STEP)

    degp = _sc_degree(dst)
    deg2 = jnp.transpose(degp)

    y1 = _t_first(x, W1, deg2)
    s1 = _sc_scatter(y1, src, dst)
    y2 = _t_mid(s1, y1, deg2, b1.reshape(1, D), W2)
    s2 = _sc_scatter(y2, src, dst)
    y3 = _t_mid(s2, y2, deg2, b2.reshape(1, D), W3)
    s3 = _sc_scatter(y3, src, dst)
    return _t_last(s3, y3, deg2, b3.reshape(1, D))


# speedup vs baseline: 30.5878x; 1.4530x over previous
import functools

import jax
import jax.numpy as jnp
from jax import lax
from jax.experimental import pallas as pl
from jax.experimental.pallas import tpu as pltpu
from jax.experimental.pallas import tpu_sc as plsc

N = 10000
E = 320000
D_IN = 128
D = 64
NC = 2
NS = 16
NW = NC * NS
STEP = 128
NSTEP = 80
EPT = NSTEP * STEP
EP = NW * EPT
NPAD = 10240
RPT = NPAD // NS

_mesh = plsc.VectorSubcoreMesh(
    core_axis_name="c", subcore_axis_name="s", num_cores=NC, num_subcores=NS
)
_sc_params = pltpu.CompilerParams(use_tc_tiling_on_sc=False)


@functools.partial(
    pl.kernel,
    out_type=jax.ShapeDtypeStruct((NC, NPAD), jnp.float32),
    mesh=_mesh,
    compiler_params=_sc_params,
    scratch_types=[
        pltpu.VMEM((NSTEP, STEP), jnp.int32),
        pltpu.VMEM((RPT,), jnp.float32),
        pltpu.VMEM_SHARED((NPAD,), jnp.float32),
    ],
)
def _sc_degree(dst_hbm, out_hbm, idx_v, ones_v, hist_s):
    c = lax.axis_index("c")
    s = lax.axis_index("s")
    wid = s * NC + c
    r0 = s * RPT

    def fill(i, _):
        ones_v[pl.ds(i * 16, 16)] = jnp.full((16,), 1.0, jnp.float32)
        return 0

    lax.fori_loop(0, RPT // 16, fill, 0)
    pltpu.sync_copy(ones_v, hist_s.at[pl.ds(r0, RPT)])
    pltpu.sync_copy(dst_hbm.at[wid], idx_v)
    plsc.subcore_barrier()

    def body(j, _):
        pltpu.sync_copy(
            ones_v.at[pl.ds(0, STEP)], hist_s.at[idx_v.at[j]], add=True
        )
        return 0

    lax.fori_loop(0, NSTEP, body, 0)
    plsc.subcore_barrier()
    pltpu.sync_copy(hist_s.at[pl.ds(r0, RPT)], out_hbm.at[c, pl.ds(r0, RPT)])


@functools.partial(
    pl.kernel,
    out_type=jax.ShapeDtypeStruct((NC, NPAD, D), jnp.float32),
    mesh=_mesh,
    compiler_params=_sc_params,
    scratch_types=[
        pltpu.VMEM((NSTEP, STEP), jnp.int32),
        pltpu.VMEM((NSTEP, STEP), jnp.int32),
        pltpu.VMEM((STEP, D), jnp.float32),
        pltpu.VMEM((STEP, D), jnp.float32),
        pltpu.VMEM_SHARED((NPAD, D), jnp.float32),
        pltpu.SemaphoreType.DMA,
        pltpu.SemaphoreType.DMA,
        pltpu.SemaphoreType.DMA,
        pltpu.SemaphoreType.DMA,
    ],
)
def _sc_scatter(y_hbm, src_hbm, dst_hbm, out_hbm, si_v, di_v, b0, b1, acc_s,
                gs0, gs1, ss0, ss1):
    c = lax.axis_index("c")
    s = lax.axis_index("s")
    wid = s * NC + c
    r0 = s * RPT

    pltpu.sync_copy(y_hbm.at[pl.ds(r0, RPT)], acc_s.at[pl.ds(r0, RPT)])
    pltpu.sync_copy(src_hbm.at[wid], si_v)
    pltpu.sync_copy(dst_hbm.at[wid], di_v)
    plsc.subcore_barrier()

    pltpu.async_copy(y_hbm.at[si_v.at[0]], b0, gs0)
    pltpu.async_copy(y_hbm.at[si_v.at[1]], b1, gs1)

    def body(j, _):
        i0 = 2 * j
        i1 = i0 + 1
        pltpu.make_async_copy(y_hbm.at[si_v.at[i0]], b0, gs0).wait()
        pltpu.async_copy(b0, acc_s.at[di_v.at[i0]], ss0, add=True)
        pltpu.make_async_copy(y_hbm.at[si_v.at[i1]], b1, gs1).wait()
        pltpu.async_copy(b1, acc_s.at[di_v.at[i1]], ss1, add=True)

        @pl.when(i0 + 2 < NSTEP)
        def _():
            pltpu.make_async_copy(b0, acc_s.at[di_v.at[i0]], ss0).wait()
            pltpu.async_copy(y_hbm.at[si_v.at[i0 + 2]], b0, gs0)

        @pl.when(i1 + 2 < NSTEP)
        def _():
            pltpu.make_async_copy(b1, acc_s.at[di_v.at[i1]], ss1).wait()
            pltpu.async_copy(y_hbm.at[si_v.at[i1 + 2]], b1, gs1)

        return 0

    lax.fori_loop(0, NSTEP // 2, body, 0)
    pltpu.make_async_copy(b0, acc_s.at[di_v.at[NSTEP - 2]], ss0).wait()
    pltpu.make_async_copy(b1, acc_s.at[di_v.at[NSTEP - 1]], ss1).wait()
    plsc.subcore_barrier()
    pltpu.sync_copy(acc_s.at[pl.ds(r0, RPT)], out_hbm.at[c, pl.ds(r0, RPT)])


def _dinv(deg2):
    return lax.rsqrt(deg2[:, 0:1] + deg2[:, 1:2] - 1.0)


def _tc_first(x_ref, w_ref, deg2_ref, y_ref):
    di = _dinv(deg2_ref[...])
    h = jnp.dot(x_ref[...], w_ref[...], preferred_element_type=jnp.float32)
    y_ref[:N, :] = h * di[:N]


def _tc_mid(sp_ref, y_ref, deg2_ref, b_ref, w_ref, o_ref):
    di = _dinv(deg2_ref[...])
    ssum = sp_ref[0] + sp_ref[1] - y_ref[...]
    z = jnp.maximum(ssum * di + b_ref[...], 0.0)
    h = jnp.dot(z[:N], w_ref[...], preferred_element_type=jnp.float32)
    o_ref[:N, :] = h * di[:N]


def _tc_last(sp_ref, y_ref, deg2_ref, b_ref, o_ref):
    di = _dinv(deg2_ref[...])
    ssum = sp_ref[0, :N] + sp_ref[1, :N] - y_ref[:N]
    o_ref[...] = jnp.maximum(ssum * di[:N] + b_ref[...], 0.0)


_t_first = pl.pallas_call(
    _tc_first, out_shape=jax.ShapeDtypeStruct((NPAD, D), jnp.float32)
)
_t_mid = pl.pallas_call(
    _tc_mid, out_shape=jax.ShapeDtypeStruct((NPAD, D), jnp.float32)
)
_t_last = pl.pallas_call(
    _tc_last, out_shape=jax.ShapeDtypeStruct((N, D), jnp.float32)
)


def kernel(x, edge_index, W1, b1, W2, b2, W3, b3):
    ei = edge_index.astype(jnp.int32)
    pad = jnp.arange(EP - E, dtype=jnp.int32)
    src = jnp.concatenate([ei[0], pad % N]).reshape(NW, NSTEP, STEP)
    dst = jnp.concatenate([ei[1], N + pad % (NPAD - N)]).reshape(
        NW, NSTEP, STEP)

    degp = _sc_degree(dst)
    deg2 = jnp.transpose(degp)

    y1 = _t_first(x, W1, deg2)
    s1 = _sc_scatter(y1, src, dst)
    y2 = _t_mid(s1, y1, deg2, b1.reshape(1, D), W2)
    s2 = _sc_scatter(y2, src, dst)
    y3 = _t_mid(s2, y2, deg2, b2.reshape(1, D), W3)
    s3 = _sc_scatter(y3, src, dst)
    return _t_last(s3, y3, deg2, b3.reshape(1, D))


# speedup vs baseline: 38.8336x; 1.2696x over previous
import functools

import jax
import jax.numpy as jnp
from jax import lax
from jax.experimental import pallas as pl
from jax.experimental.pallas import tpu as pltpu
from jax.experimental.pallas import tpu_sc as plsc

N = 10000
E = 320000
D_IN = 128
D = 64
NC = 2
NS = 16
NW = NC * NS
STEP = 128
NSTEP = 80
EPT = NSTEP * STEP
EP = NW * EPT
NPAD = 10240
RPT = NPAD // NS

_mesh = plsc.VectorSubcoreMesh(
    core_axis_name="c", subcore_axis_name="s", num_cores=NC, num_subcores=NS
)
_sc_params = pltpu.CompilerParams(use_tc_tiling_on_sc=False)


@functools.partial(
    pl.kernel,
    out_type=jax.ShapeDtypeStruct((NC, NPAD), jnp.float32),
    mesh=_mesh,
    compiler_params=_sc_params,
    scratch_types=[
        pltpu.VMEM((NSTEP, STEP), jnp.int32),
        pltpu.VMEM((RPT,), jnp.float32),
        pltpu.VMEM_SHARED((NPAD,), jnp.float32),
    ],
)
def _sc_degree(dst_hbm, out_hbm, idx_v, ones_v, hist_s):
    c = lax.axis_index("c")
    s = lax.axis_index("s")
    wid = s * NC + c
    r0 = s * RPT

    def fill(i, _):
        ones_v[pl.ds(i * 16, 16)] = jnp.full((16,), 1.0, jnp.float32)
        return 0

    lax.fori_loop(0, RPT // 16, fill, 0)
    pltpu.sync_copy(ones_v, hist_s.at[pl.ds(r0, RPT)])
    pltpu.sync_copy(dst_hbm.at[wid], idx_v)
    plsc.subcore_barrier()

    def body(j, _):
        pltpu.sync_copy(
            ones_v.at[pl.ds(0, STEP)], hist_s.at[idx_v.at[j]], add=True
        )
        return 0

    lax.fori_loop(0, NSTEP, body, 0)
    plsc.subcore_barrier()
    pltpu.sync_copy(hist_s.at[pl.ds(r0, RPT)], out_hbm.at[c, pl.ds(r0, RPT)])


@functools.partial(
    pl.kernel,
    out_type=jax.ShapeDtypeStruct((NC, NPAD, D), jnp.float32),
    mesh=_mesh,
    compiler_params=_sc_params,
    scratch_types=[
        pltpu.VMEM((NSTEP, STEP), jnp.int32),
        pltpu.VMEM((NSTEP, STEP), jnp.int32),
        pltpu.VMEM((STEP, D), jnp.float32),
        pltpu.VMEM((STEP, D), jnp.float32),
        pltpu.VMEM((STEP, D), jnp.float32),
        pltpu.VMEM((STEP, D), jnp.float32),
        pltpu.VMEM_SHARED((NPAD, D), jnp.float32),
        pltpu.SemaphoreType.DMA,
        pltpu.SemaphoreType.DMA,
        pltpu.SemaphoreType.DMA,
        pltpu.SemaphoreType.DMA,
    ],
)
def _sc_scatter(y_hbm, src_hbm, dst_hbm, out_hbm, si_v, di_v,
                b0, b1, b2, b3, acc_s, s0, s1, s2, s3):
    c = lax.axis_index("c")
    s = lax.axis_index("s")
    wid = s * NC + c
    r0 = s * RPT
    bufs = (b0, b1, b2, b3)
    sems = (s0, s1, s2, s3)

    pltpu.sync_copy(y_hbm.at[pl.ds(r0, RPT)], acc_s.at[pl.ds(r0, RPT)])
    pltpu.sync_copy(src_hbm.at[wid], si_v)
    pltpu.sync_copy(dst_hbm.at[wid], di_v)
    plsc.subcore_barrier()

    pltpu.async_copy(y_hbm.at[si_v.at[0]], b0, s0)
    pltpu.async_copy(y_hbm.at[si_v.at[1]], b1, s1)

    def step(i, k):
        buf, sem = bufs[k], sems[k]
        nbuf, nsem = bufs[(k + 2) % 4], sems[(k + 2) % 4]

        @pl.when(i + 2 < NSTEP)
        def _():
            @pl.when(i >= 2)
            def _():
                pltpu.make_async_copy(
                    nbuf, acc_s.at[di_v.at[i - 2]], nsem).wait()

            pltpu.async_copy(y_hbm.at[si_v.at[i + 2]], nbuf, nsem)

        pltpu.make_async_copy(y_hbm.at[si_v.at[i]], buf, sem).wait()
        pltpu.async_copy(buf, acc_s.at[di_v.at[i]], sem, add=True)

    def body(j, _):
        base = 4 * j
        for k in range(4):
            step(base + k, k)
        return 0

    lax.fori_loop(0, NSTEP // 4, body, 0)
    for t in range(NSTEP - 4, NSTEP):
        pltpu.make_async_copy(
            bufs[t % 4], acc_s.at[di_v.at[t]], sems[t % 4]).wait()
    plsc.subcore_barrier()
    pltpu.sync_copy(acc_s.at[pl.ds(r0, RPT)], out_hbm.at[c, pl.ds(r0, RPT)])


def _dinv(deg2):
    return lax.rsqrt(deg2[:, 0:1] + deg2[:, 1:2] - 1.0)


def _tc_first(x_ref, w_ref, deg2_ref, y_ref):
    di = _dinv(deg2_ref[...])
    h = jnp.dot(x_ref[...], w_ref[...], preferred_element_type=jnp.float32)
    y_ref[:N, :] = h * di[:N]


def _tc_mid(sp_ref, y_ref, deg2_ref, b_ref, w_ref, o_ref):
    di = _dinv(deg2_ref[...])
    ssum = sp_ref[0] + sp_ref[1] - y_ref[...]
    z = jnp.maximum(ssum * di + b_ref[...], 0.0)
    h = jnp.dot(z[:N], w_ref[...], preferred_element_type=jnp.float32)
    o_ref[:N, :] = h * di[:N]


def _tc_last(sp_ref, y_ref, deg2_ref, b_ref, o_ref):
    di = _dinv(deg2_ref[...])
    ssum = sp_ref[0, :N] + sp_ref[1, :N] - y_ref[:N]
    o_ref[...] = jnp.maximum(ssum * di[:N] + b_ref[...], 0.0)


_t_first = pl.pallas_call(
    _tc_first, out_shape=jax.ShapeDtypeStruct((NPAD, D), jnp.float32)
)
_t_mid = pl.pallas_call(
    _tc_mid, out_shape=jax.ShapeDtypeStruct((NPAD, D), jnp.float32)
)
_t_last = pl.pallas_call(
    _tc_last, out_shape=jax.ShapeDtypeStruct((N, D), jnp.float32)
)


def kernel(x, edge_index, W1, b1, W2, b2, W3, b3):
    ei = edge_index.astype(jnp.int32)
    pad = jnp.arange(EP - E, dtype=jnp.int32)
    src = jnp.concatenate([ei[0], pad % N]).reshape(NW, NSTEP, STEP)
    dst = jnp.concatenate([ei[1], N + pad % (NPAD - N)]).reshape(
        NW, NSTEP, STEP)

    degp = _sc_degree(dst)
    deg2 = jnp.transpose(degp)

    y1 = _t_first(x, W1, deg2)
    s1 = _sc_scatter(y1, src, dst)
    y2 = _t_mid(s1, y1, deg2, b1.reshape(1, D), W2)
    s2 = _sc_scatter(y2, src, dst)
    y3 = _t_mid(s2, y2, deg2, b2.reshape(1, D), W3)
    s3 = _sc_scatter(y3, src, dst)
    return _t_last(s3, y3, deg2, b3.reshape(1, D))


# speedup vs baseline: 39.7978x; 1.0248x over previous
import functools

import jax
import jax.numpy as jnp
from jax import lax
from jax.experimental import pallas as pl
from jax.experimental.pallas import tpu as pltpu
from jax.experimental.pallas import tpu_sc as plsc

N = 10000
E = 320000
D_IN = 128
D = 64
NC = 2
NS = 16
NW = NC * NS
STEP = 128
NSTEP = 80
EPT = NSTEP * STEP
EP = NW * EPT
NPAD = 10240
RPT = NPAD // NS
NBUF = 8
LEAD = 4

_mesh = plsc.VectorSubcoreMesh(
    core_axis_name="c", subcore_axis_name="s", num_cores=NC, num_subcores=NS
)
_sc_params = pltpu.CompilerParams(use_tc_tiling_on_sc=False)


@functools.partial(
    pl.kernel,
    out_type=jax.ShapeDtypeStruct((NC, NPAD), jnp.float32),
    mesh=_mesh,
    compiler_params=_sc_params,
    scratch_types=[
        pltpu.VMEM((NSTEP, STEP), jnp.int32),
        pltpu.VMEM((RPT,), jnp.float32),
        pltpu.VMEM_SHARED((NPAD,), jnp.float32),
    ],
)
def _sc_degree(dst_hbm, out_hbm, idx_v, ones_v, hist_s):
    c = lax.axis_index("c")
    s = lax.axis_index("s")
    wid = s * NC + c
    r0 = s * RPT

    def fill(i, _):
        ones_v[pl.ds(i * 16, 16)] = jnp.full((16,), 1.0, jnp.float32)
        return 0

    lax.fori_loop(0, RPT // 16, fill, 0)
    pltpu.sync_copy(ones_v, hist_s.at[pl.ds(r0, RPT)])
    pltpu.sync_copy(dst_hbm.at[wid], idx_v)
    plsc.subcore_barrier()

    def body(j, _):
        pltpu.sync_copy(
            ones_v.at[pl.ds(0, STEP)], hist_s.at[idx_v.at[j]], add=True
        )
        return 0

    lax.fori_loop(0, NSTEP, body, 0)
    plsc.subcore_barrier()
    pltpu.sync_copy(hist_s.at[pl.ds(r0, RPT)], out_hbm.at[c, pl.ds(r0, RPT)])


@functools.partial(
    pl.kernel,
    out_type=jax.ShapeDtypeStruct((NC, NPAD, D), jnp.float32),
    mesh=_mesh,
    compiler_params=_sc_params,
    scratch_types=[
        pltpu.VMEM((NSTEP, STEP), jnp.int32),
        pltpu.VMEM((NSTEP, STEP), jnp.int32),
        [pltpu.VMEM((STEP, D), jnp.float32)] * NBUF,
        pltpu.VMEM_SHARED((NPAD, D), jnp.float32),
        [pltpu.SemaphoreType.DMA] * NBUF,
    ],
)
def _sc_scatter(y_hbm, src_hbm, dst_hbm, out_hbm, si_v, di_v,
                bufs, acc_s, sems):
    c = lax.axis_index("c")
    s = lax.axis_index("s")
    wid = s * NC + c
    r0 = s * RPT

    pltpu.sync_copy(y_hbm.at[pl.ds(r0, RPT)], acc_s.at[pl.ds(r0, RPT)])
    pltpu.sync_copy(src_hbm.at[wid], si_v)
    pltpu.sync_copy(dst_hbm.at[wid], di_v)
    plsc.subcore_barrier()

    for i in range(LEAD):
        pltpu.async_copy(y_hbm.at[si_v.at[i]], bufs[i], sems[i])

    def step(i, k):
        nk = (k + LEAD) % NBUF

        @pl.when(i + LEAD < NSTEP)
        def _():
            @pl.when(i + LEAD >= NBUF)
            def _():
                pltpu.make_async_copy(
                    bufs[nk], acc_s.at[di_v.at[i + LEAD - NBUF]],
                    sems[nk]).wait()

            pltpu.async_copy(y_hbm.at[si_v.at[i + LEAD]], bufs[nk], sems[nk])

        pltpu.make_async_copy(y_hbm.at[si_v.at[i]], bufs[k], sems[k]).wait()
        pltpu.async_copy(bufs[k], acc_s.at[di_v.at[i]], sems[k], add=True)

    def body(j, _):
        base = NBUF * j
        for k in range(NBUF):
            step(base + k, k)
        return 0

    lax.fori_loop(0, NSTEP // NBUF, body, 0)
    for t in range(NSTEP - NBUF, NSTEP):
        pltpu.make_async_copy(
            bufs[t % NBUF], acc_s.at[di_v.at[t]], sems[t % NBUF]).wait()
    plsc.subcore_barrier()
    pltpu.sync_copy(acc_s.at[pl.ds(r0, RPT)], out_hbm.at[c, pl.ds(r0, RPT)])


def _dinv(deg2):
    return lax.rsqrt(deg2[:, 0:1] + deg2[:, 1:2] - 1.0)


def _tc_first(x_ref, w_ref, deg2_ref, y_ref):
    di = _dinv(deg2_ref[...])
    h = jnp.dot(x_ref[...], w_ref[...], preferred_element_type=jnp.float32)
    y_ref[:N, :] = h * di[:N]


def _tc_mid(sp_ref, y_ref, deg2_ref, b_ref, w_ref, o_ref):
    di = _dinv(deg2_ref[...])
    ssum = sp_ref[0] + sp_ref[1] - y_ref[...]
    z = jnp.maximum(ssum * di + b_ref[...], 0.0)
    h = jnp.dot(z[:N], w_ref[...], preferred_element_type=jnp.float32)
    o_ref[:N, :] = h * di[:N]


def _tc_last(sp_ref, y_ref, deg2_ref, b_ref, o_ref):
    di = _dinv(deg2_ref[...])
    ssum = sp_ref[0, :N] + sp_ref[1, :N] - y_ref[:N]
    o_ref[...] = jnp.maximum(ssum * di[:N] + b_ref[...], 0.0)


_t_first = pl.pallas_call(
    _tc_first, out_shape=jax.ShapeDtypeStruct((NPAD, D), jnp.float32)
)
_t_mid = pl.pallas_call(
    _tc_mid, out_shape=jax.ShapeDtypeStruct((NPAD, D), jnp.float32)
)
_t_last = pl.pallas_call(
    _tc_last, out_shape=jax.ShapeDtypeStruct((N, D), jnp.float32)
)


def kernel(x, edge_index, W1, b1, W2, b2, W3, b3):
    ei = edge_index.astype(jnp.int32)
    pad = jnp.arange(EP - E, dtype=jnp.int32)
    src = jnp.concatenate([ei[0], pad % N]).reshape(NW, NSTEP, STEP)
    dst = jnp.concatenate([ei[1], N + pad % (NPAD - N)]).reshape(
        NW, NSTEP, STEP)

    degp = _sc_degree(dst)
    deg2 = jnp.transpose(degp)

    y1 = _t_first(x, W1, deg2)
    s1 = _sc_scatter(y1, src, dst)
    y2 = _t_mid(s1, y1, deg2, b1.reshape(1, D), W2)
    s2 = _sc_scatter(y2, src, dst)
    y3 = _t_mid(s2, y2, deg2, b2.reshape(1, D), W3)
    s3 = _sc_scatter(y3, src, dst)
    return _t_last(s3, y3, deg2, b3.reshape(1, D))


# speedup vs baseline: 43.7573x; 1.0995x over previous
import functools

import jax
import jax.numpy as jnp
from jax import lax
from jax.experimental import pallas as pl
from jax.experimental.pallas import tpu as pltpu
from jax.experimental.pallas import tpu_sc as plsc

N = 10000
E = 320000
D_IN = 128
D = 64
NC = 2
NS = 16
NW = NC * NS
STEP = 128
NSTEP = 80
EPT = NSTEP * STEP
EP = NW * EPT
NPAD = 10240
RPT = NPAD // NS
NBUF = 8
LEAD = 4

_mesh = plsc.VectorSubcoreMesh(
    core_axis_name="c", subcore_axis_name="s", num_cores=NC, num_subcores=NS
)
_sc_params = pltpu.CompilerParams(use_tc_tiling_on_sc=False)


@functools.partial(
    pl.kernel,
    out_type=jax.ShapeDtypeStruct((NC, NPAD), jnp.float32),
    mesh=_mesh,
    compiler_params=_sc_params,
    scratch_types=[
        pltpu.VMEM((NSTEP, STEP), jnp.int32),
        pltpu.VMEM((RPT,), jnp.float32),
        pltpu.VMEM_SHARED((NPAD,), jnp.float32),
    ],
)
def _sc_degree(dst_hbm, out_hbm, idx_v, ones_v, hist_s):
    c = lax.axis_index("c")
    s = lax.axis_index("s")
    wid = s * NC + c
    r0 = s * RPT

    def fill(i, _):
        ones_v[pl.ds(i * 16, 16)] = jnp.full((16,), 1.0, jnp.float32)
        return 0

    lax.fori_loop(0, RPT // 16, fill, 0)
    pltpu.sync_copy(ones_v, hist_s.at[pl.ds(r0, RPT)])
    pltpu.sync_copy(dst_hbm.at[wid], idx_v)
    plsc.subcore_barrier()

    def body(j, _):
        pltpu.sync_copy(
            ones_v.at[pl.ds(0, STEP)], hist_s.at[idx_v.at[j]], add=True
        )
        return 0

    lax.fori_loop(0, NSTEP, body, 0)
    plsc.subcore_barrier()
    pltpu.sync_copy(hist_s.at[pl.ds(r0, RPT)], out_hbm.at[c, pl.ds(r0, RPT)])


@functools.partial(
    pl.kernel,
    out_type=jax.ShapeDtypeStruct((NC, NPAD, D), jnp.float32),
    mesh=_mesh,
    compiler_params=_sc_params,
    scratch_types=[
        pltpu.VMEM((NSTEP, STEP), jnp.int32),
        pltpu.VMEM((NSTEP, STEP), jnp.int32),
        [pltpu.VMEM((STEP, D), jnp.float32)] * NBUF,
        pltpu.VMEM_SHARED((NPAD, D), jnp.float32),
        [pltpu.SemaphoreType.DMA] * NBUF,
    ],
)
def _sc_scatter(y_hbm, src_hbm, dst_hbm, out_hbm, si_v, di_v,
                bufs, acc_s, sems):
    c = lax.axis_index("c")
    s = lax.axis_index("s")
    wid = s * NC + c
    r0 = s * RPT

    pltpu.sync_copy(y_hbm.at[pl.ds(r0, RPT)], acc_s.at[pl.ds(r0, RPT)])
    pltpu.sync_copy(src_hbm.at[wid], si_v)
    pltpu.sync_copy(dst_hbm.at[wid], di_v)
    plsc.subcore_barrier()

    for i in range(LEAD):
        pltpu.async_copy(y_hbm.at[si_v.at[i]], bufs[i], sems[i])

    def step(i, k):
        nk = (k + LEAD) % NBUF

        @pl.when(i + LEAD < NSTEP)
        def _():
            @pl.when(i + LEAD >= NBUF)
            def _():
                pltpu.make_async_copy(
                    bufs[nk], acc_s.at[di_v.at[i + LEAD - NBUF]],
                    sems[nk]).wait()

            pltpu.async_copy(y_hbm.at[si_v.at[i + LEAD]], bufs[nk], sems[nk])

        pltpu.make_async_copy(y_hbm.at[si_v.at[i]], bufs[k], sems[k]).wait()
        pltpu.async_copy(bufs[k], acc_s.at[di_v.at[i]], sems[k], add=True)

    def body(j, _):
        base = NBUF * j
        for k in range(NBUF):
            step(base + k, k)
        return 0

    lax.fori_loop(0, NSTEP // NBUF, body, 0)
    for t in range(NSTEP - NBUF, NSTEP):
        pltpu.make_async_copy(
            bufs[t % NBUF], acc_s.at[di_v.at[t]], sems[t % NBUF]).wait()
    plsc.subcore_barrier()
    pltpu.sync_copy(acc_s.at[pl.ds(r0, RPT)], out_hbm.at[c, pl.ds(r0, RPT)])


def _dinv(deg2):
    return lax.rsqrt(deg2[:, 0:1] + deg2[:, 1:2] - 1.0)


NH = NPAD // 2


def _unpair(p):
    return jnp.concatenate([p[:, :D], p[:, D:]], axis=0)


def _store_paired(o_ref, y):
    o_ref[:, :D] = y[:NH]
    o_ref[: N - NH, D:] = y[NH:]


def _tc_first(x_ref, w_ref, deg2_ref, y_ref):
    di = _dinv(deg2_ref[...])
    h = jnp.dot(x_ref[...], w_ref[...], preferred_element_type=jnp.float32)
    _store_paired(y_ref, h * di[:N])


def _tc_mid(spp_ref, yp_ref, deg2_ref, b_ref, w_ref, o_ref):
    di = _dinv(deg2_ref[...])
    ssum = _unpair(spp_ref[0] + spp_ref[1] - yp_ref[...])
    z = jnp.maximum(ssum * di + b_ref[...], 0.0)
    h = jnp.dot(z[:N], w_ref[...], preferred_element_type=jnp.float32)
    _store_paired(o_ref, h * di[:N])


def _tc_last(spp_ref, yp_ref, deg2_ref, b_ref, o_ref):
    di = _dinv(deg2_ref[...])
    ssum = _unpair(spp_ref[0] + spp_ref[1] - yp_ref[...])[:N]
    o_ref[...] = jnp.maximum(ssum * di[:N] + b_ref[...], 0.0)


_pair = jax.ShapeDtypeStruct((NH, 2 * D), jnp.float32)
_t_first = pl.pallas_call(_tc_first, out_shape=_pair)
_t_mid = pl.pallas_call(_tc_mid, out_shape=_pair)
_t_last = pl.pallas_call(
    _tc_last, out_shape=jax.ShapeDtypeStruct((N, D), jnp.float32)
)


def kernel(x, edge_index, W1, b1, W2, b2, W3, b3):
    ei = edge_index.astype(jnp.int32)
    sv = jnp.where(ei[0] < NH, 2 * ei[0], 2 * ei[0] - (NPAD - 1))
    dv = jnp.where(ei[1] < NH, 2 * ei[1], 2 * ei[1] - (NPAD - 1))
    pad = jnp.arange(EP - E, dtype=jnp.int32)
    src = jnp.concatenate([sv, 2 * (pad % NH)]).reshape(NW, NSTEP, STEP)
    dst = jnp.concatenate(
        [dv, 2 * (N - NH) + 1 + 2 * (pad % (NPAD - N))]
    ).reshape(NW, NSTEP, STEP)

    degp = _sc_degree(dst)
    tau = jnp.concatenate(
        [2 * jnp.arange(NH, dtype=jnp.int32),
         2 * jnp.arange(NPAD - NH, dtype=jnp.int32) + 1])
    deg2 = jnp.transpose(degp)[tau]

    def scatter(yp):
        sp = _sc_scatter(yp.reshape(NPAD, D), src, dst)
        return sp.reshape(NC, NH, 2 * D)

    y1 = _t_first(x, W1, deg2)
    y2 = _t_mid(scatter(y1), y1, deg2, b1.reshape(1, D), W2)
    y3 = _t_mid(scatter(y2), y2, deg2, b2.reshape(1, D), W3)
    return _t_last(scatter(y3), y3, deg2, b3.reshape(1, D))
